# baseline (speedup 1.0000x reference)
import functools

import jax
import jax.numpy as jnp
from jax import lax
from jax.experimental import pallas as pl
from jax.experimental.pallas import tpu as pltpu
from jax.experimental.pallas import tpu_sc as plsc

NC = 2
NS = 16
L = 16
NW = NC * NS


def _butterfly_kernel(B, C):
    per_w = B // NW
    n_chunks = per_w // C
    mesh = plsc.VectorSubcoreMesh(
        core_axis_name="c", subcore_axis_name="s", num_cores=NC,
        num_subcores=NS)
    f32 = jnp.float32
    out_sds = jax.ShapeDtypeStruct((B,), f32)

    @functools.partial(
        pl.kernel,
        mesh=mesh,
        out_type=(out_sds, out_sds, out_sds, out_sds),
        scratch_types=[
            pltpu.VMEM((C,), jnp.int32),
            pltpu.VMEM((C,), jnp.int32),
            pltpu.VMEM((C,), f32),
            pltpu.VMEM((C,), f32),
            pltpu.VMEM((C,), f32),
            pltpu.VMEM((C,), f32),
            pltpu.VMEM((C,), f32),
            pltpu.VMEM((C,), f32),
            pltpu.SemaphoreType.DMA,
            pltpu.SemaphoreType.DMA,
            pltpu.SemaphoreType.DMA,
            pltpu.SemaphoreType.DMA,
            pltpu.SemaphoreType.DMA,
            pltpu.SemaphoreType.DMA,
        ],
    )
    def k(ar_h, ai_h, br_h, bi_h, idx_h, twf_h,
          o1r_h, o1i_h, o2r_h, o2i_h,
          idx_v, idxr_v, wr_v, wi_v, ar_v, ai_v, br_v, bi_v,
          sem_idx, sem_ar, sem_ai, sem_br, sem_bi, sem_w):
        wid = lax.axis_index("s") * NC + lax.axis_index("c")
        base_w = pl.multiple_of(wid * per_w, per_w)

        for ci in range(n_chunks):
            base = base_w + ci * C
            cp_idx = pltpu.async_copy(idx_h.at[pl.ds(base, C)], idx_v,
                                      sem_idx)
            cp_ar = pltpu.async_copy(ar_h.at[pl.ds(base, C)], ar_v, sem_ar)
            cp_ai = pltpu.async_copy(ai_h.at[pl.ds(base, C)], ai_v, sem_ai)
            cp_br = pltpu.async_copy(br_h.at[pl.ds(base, C)], br_v, sem_br)
            cp_bi = pltpu.async_copy(bi_h.at[pl.ds(base, C)], bi_v, sem_bi)
            cp_idx.wait()

            def dbl_body(g, carry):
                s = pl.multiple_of(g * L, L)
                v2 = idx_v[pl.ds(s, L)] * 2
                idxr_v[pl.ds(s, L)] = v2
                idx_v[pl.ds(s, L)] = v2 + 1
                return carry

            lax.fori_loop(0, C // L, dbl_body, 0)
            cp_wr = pltpu.async_copy(twf_h.at[idxr_v], wr_v, sem_w)
            cp_wi = pltpu.async_copy(twf_h.at[idx_v], wi_v, sem_idx)
            cp_ar.wait()
            cp_ai.wait()
            cp_br.wait()
            cp_bi.wait()
            cp_wr.wait()
            cp_wi.wait()

            def g_body(g, carry):
                s = pl.multiple_of(g * L, L)
                wr = wr_v[pl.ds(s, L)]
                wi = wi_v[pl.ds(s, L)]
                ar = ar_v[pl.ds(s, L)]
                ai = ai_v[pl.ds(s, L)]
                br = br_v[pl.ds(s, L)]
                bi = bi_v[pl.ds(s, L)]
                wbr = wr * br - wi * bi
                wbi = wr * bi + wi * br
                ar_v[pl.ds(s, L)] = ar + wbr
                ai_v[pl.ds(s, L)] = ai + wbi
                br_v[pl.ds(s, L)] = ar - wbr
                bi_v[pl.ds(s, L)] = ai - wbi
                return carry

            lax.fori_loop(0, C // L, g_body, 0)

            pltpu.sync_copy(ar_v, o1r_h.at[pl.ds(base, C)])
            pltpu.sync_copy(ai_v, o1i_h.at[pl.ds(base, C)])
            pltpu.sync_copy(br_v, o2r_h.at[pl.ds(base, C)])
            pltpu.sync_copy(bi_v, o2i_h.at[pl.ds(base, C)])

    return k


def kernel(a_real, a_imag, b_real, b_imag, twiddle_idx, twiddle_factors):
    B = a_real.shape[0]
    k = _butterfly_kernel(B, 8192)
    flat = twiddle_factors.reshape(-1)
    pad = (-flat.shape[0]) % 1024
    flat = jnp.pad(flat, (0, pad))
    return k(a_real, a_imag, b_real, b_imag,
             twiddle_idx.astype(jnp.int32), flat)


# speedup vs baseline: 11.7238x; 11.7238x over previous
import functools

import jax
import jax.numpy as jnp
from jax import lax
from jax.experimental import pallas as pl
from jax.experimental.pallas import tpu as pltpu
from jax.experimental.pallas import tpu_sc as plsc

NC = 2
NS = 16
L = 16
NW = NC * NS
NSETS = 3
UNROLL = 4


def _butterfly_kernel(B, C, N):
    per_w = B // NW
    n_chunks = per_w // C
    mesh = plsc.VectorSubcoreMesh(
        core_axis_name="c", subcore_axis_name="s", num_cores=NC,
        num_subcores=NS)
    f32 = jnp.float32
    out_sds = jax.ShapeDtypeStruct((B,), f32)

    buf_types = []
    for _ in range(NSETS):
        buf_types += [pltpu.VMEM((C,), jnp.int32),
                      pltpu.VMEM((C,), jnp.int32)]
        buf_types += [pltpu.VMEM((C,), f32) for _ in range(6)]
    sem_types = [pltpu.SemaphoreType.DMA] * (4 * NSETS)

    @functools.partial(
        pl.kernel,
        mesh=mesh,
        out_type=(out_sds, out_sds, out_sds, out_sds),
        scratch_types=buf_types + sem_types,
    )
    def k(ar_h, ai_h, br_h, bi_h, idx_h, twf_h,
          o1r_h, o1i_h, o2r_h, o2i_h, *scratch):
        bufs = [scratch[8 * s:8 * (s + 1)] for s in range(NSETS)]
        sems = scratch[8 * NSETS:]
        sem_idx = sems[0:NSETS]
        sem_in = sems[NSETS:2 * NSETS]
        sem_w = sems[2 * NSETS:3 * NSETS]
        sem_out = sems[3 * NSETS:4 * NSETS]

        wid = lax.axis_index("s") * NC + lax.axis_index("c")
        base_w = pl.multiple_of(wid * per_w, per_w)

        pend = {}

        def issue_in(c):
            s = c % NSETS
            idx_v, _, _, _, ar_v, ai_v, br_v, bi_v = bufs[s]
            base = base_w + c * C
            ds = pl.ds(base, C)
            cp_i = pltpu.async_copy(idx_h.at[ds], idx_v, sem_idx[s])
            cps = [pltpu.async_copy(ar_h.at[ds], ar_v, sem_in[s]),
                   pltpu.async_copy(ai_h.at[ds], ai_v, sem_in[s]),
                   pltpu.async_copy(br_h.at[ds], br_v, sem_in[s]),
                   pltpu.async_copy(bi_h.at[ds], bi_v, sem_in[s])]
            pend[("idx", c)] = cp_i
            pend[("in", c)] = cps

        def offsets_and_gather(c):
            s = c % NSETS
            idx_v, idxi_v, wr_v, wi_v = bufs[s][:4]
            pend[("idx", c)].wait()

            def off_body(g, carry):
                for u in range(UNROLL):
                    p = pl.multiple_of(g * (L * UNROLL) + u * L, L)
                    idxi_v[pl.ds(p, L)] = idx_v[pl.ds(p, L)] + N
                return carry

            lax.fori_loop(0, C // (L * UNROLL), off_body, 0)
            pend[("w", c)] = [
                pltpu.async_copy(twf_h.at[idx_v], wr_v, sem_w[s]),
                pltpu.async_copy(twf_h.at[idxi_v], wi_v, sem_w[s])]

        def compute(c):
            s = c % NSETS
            _, _, wr_v, wi_v, ar_v, ai_v, br_v, bi_v = bufs[s]
            for cp in pend.pop(("in", c)):
                cp.wait()
            for cp in pend.pop(("w", c)):
                cp.wait()

            def g_body(g, carry):
                for u in range(UNROLL):
                    p = pl.multiple_of(g * (L * UNROLL) + u * L, L)
                    sl = pl.ds(p, L)
                    wr = wr_v[sl]
                    wi = wi_v[sl]
                    ar = ar_v[sl]
                    ai = ai_v[sl]
                    br = br_v[sl]
                    bi = bi_v[sl]
                    wbr = wr * br - wi * bi
                    wbi = wr * bi + wi * br
                    ar_v[sl] = ar + wbr
                    ai_v[sl] = ai + wbi
                    br_v[sl] = ar - wbr
                    bi_v[sl] = ai - wbi
                return carry

            lax.fori_loop(0, C // (L * UNROLL), g_body, 0)

        def issue_out(c):
            s = c % NSETS
            _, _, _, _, ar_v, ai_v, br_v, bi_v = bufs[s]
            base = base_w + c * C
            ds = pl.ds(base, C)
            pend[("out", c)] = [
                pltpu.async_copy(ar_v, o1r_h.at[ds], sem_out[s]),
                pltpu.async_copy(ai_v, o1i_h.at[ds], sem_out[s]),
                pltpu.async_copy(br_v, o2r_h.at[ds], sem_out[s]),
                pltpu.async_copy(bi_v, o2i_h.at[ds], sem_out[s])]

        def wait_out(c):
            for cp in pend.pop(("out", c), []):
                cp.wait()

        issue_in(0)
        if n_chunks > 1:
            issue_in(1)
        offsets_and_gather(0)
        for c in range(n_chunks):
            if c + 2 < n_chunks:
                wait_out(c - 1)
                issue_in(c + 2)
            if c + 1 < n_chunks:
                offsets_and_gather(c + 1)
            compute(c)
            issue_out(c)
        wait_out(n_chunks - 2)
        wait_out(n_chunks - 1)

    return k


def kernel(a_real, a_imag, b_real, b_imag, twiddle_idx, twiddle_factors):
    B = a_real.shape[0]
    k = _butterfly_kernel(B, 4096, twiddle_factors.shape[0])
    flat = twiddle_factors.T.reshape(-1)
    pad = (-flat.shape[0]) % 1024
    flat = jnp.pad(flat, (0, pad))
    return k(a_real, a_imag, b_real, b_imag,
             twiddle_idx.astype(jnp.int32), flat)
